# baseline (device time: 54597 ns/iter reference)
import jax
import jax.numpy as jnp
from jax import lax
from jax.experimental import pallas as pl
from jax.experimental.pallas import tpu as pltpu

N_DEV = 8


def kernel(x, w_mat):
    m, k_per = x.shape
    k, n = w_mat.shape
    m_per = m // N_DEV

    def body(x_hbm, w_hbm, out_ref, xland_ref, stage_ref, recv_ref,
             relay_ref, w_vmem, amax_ref, xdma_sems, wdma_sems,
             send_sems, frecv_sems, rrecv_sems,
             amax_send_sems, amax_recv_sems):
        my = lax.axis_index("i")
        p1 = my ^ 1
        p3 = my ^ 3
        p4 = my ^ 4

        barrier_sem = pltpu.get_barrier_semaphore()
        for d in range(1, N_DEV):
            pl.semaphore_signal(
                barrier_sem, inc=1,
                device_id=(lax.rem(my + d, N_DEV),),
                device_id_type=pl.DeviceIdType.MESH,
            )

        dests = [my ^ 2, my ^ 7, my ^ 5, my ^ 6, my, p1, p4, p3]

        def xcopy(i):
            return pltpu.make_async_copy(
                x_hbm.at[pl.ds(dests[i] * m_per, m_per), :],
                xland_ref.at[i % 4],
                xdma_sems.at[i % 4],
            )

        def wcopy(s):
            return pltpu.make_async_copy(
                w_hbm.at[pl.ds(s * k_per, k_per), :],
                w_vmem.at[pl.ds(s * k_per, k_per), :],
                wdma_sems.at[s],
            )

        for i in range(4):
            xcopy(i).start()
        for v in (0, 2, 7, 5, 4, 1, 3, 6):
            wcopy(my ^ v).start()

        pl.semaphore_wait(barrier_sem, N_DEV - 1)

        def send(sid, src, dst_dev, dst_ref, sem):
            rdma = pltpu.make_async_remote_copy(
                src_ref=src,
                dst_ref=dst_ref,
                send_sem=send_sems.at[sid],
                recv_sem=sem,
                device_id=(dst_dev,),
                device_id_type=pl.DeviceIdType.MESH,
            )
            rdma.start()
            return rdma

        def wait_relay(j):
            pltpu.make_async_remote_copy(
                src_ref=relay_ref.at[j], dst_ref=relay_ref.at[j],
                send_sem=send_sems.at[0], recv_sem=rrecv_sems.at[j],
                device_id=(my,), device_id_type=pl.DeviceIdType.MESH,
            ).wait_recv()

        def wait_final(v):
            pltpu.make_async_remote_copy(
                src_ref=recv_ref.at[v], dst_ref=recv_ref.at[v],
                send_sem=send_sems.at[0], recv_sem=frecv_sems.at[v],
                device_id=(my,), device_id_type=pl.DeviceIdType.MESH,
            ).wait_recv()

        def gemm(v):
            s = my ^ v
            wcopy(s).wait()
            out_ref[:, :] = out_ref[:, :] + jnp.dot(
                recv_ref[v],
                w_vmem[pl.ds(s * k_per, k_per), :],
                preferred_element_type=jnp.float32,
            )

        def land(i):
            xcopy(i).wait()
            stage_ref[i, :, :] = xland_ref[i % 4].astype(jnp.bfloat16)
            if i + 4 < 8:
                xcopy(i + 4).start()

        sends = []
        land(0)
        sends.append(send(1, stage_ref.at[0], p1,
                          relay_ref.at[0], rrecv_sems.at[0]))
        land(1)
        sends.append(send(2, stage_ref.at[1], p3,
                          relay_ref.at[2], rrecv_sems.at[2]))
        land(2)
        sends.append(send(3, stage_ref.at[2], p4,
                          relay_ref.at[1], rrecv_sems.at[1]))
        land(3)
        sends.append(send(4, stage_ref.at[3], my ^ 6,
                          recv_ref.at[6], frecv_sems.at[6]))
        land(4)

        wait_relay(0)
        sends.append(send(5, relay_ref.at[0], p3,
                          recv_ref.at[2], frecv_sems.at[2]))
        wait_relay(1)
        sends.append(send(6, relay_ref.at[1], p1,
                          recv_ref.at[5], frecv_sems.at[5]))
        wait_relay(2)
        sends.append(send(7, relay_ref.at[2], p4,
                          recv_ref.at[7], frecv_sems.at[7]))

        land(5)
        sends.append(send(8, stage_ref.at[5], p1,
                          recv_ref.at[1], frecv_sems.at[1]))
        land(6)
        sends.append(send(9, stage_ref.at[6], p4,
                          recv_ref.at[4], frecv_sems.at[4]))
        land(7)
        sends.append(send(11, stage_ref.at[7], p3,
                          recv_ref.at[3], frecv_sems.at[3]))

        wcopy(my).wait()
        out_ref[:, :] = jnp.dot(
            stage_ref[4],
            w_vmem[pl.ds(my * k_per, k_per), :],
            preferred_element_type=jnp.float32,
        )

        for v in (2, 7, 5, 4, 1, 3, 6):
            wait_final(v)
            gemm(v)

        for rdma in sends:
            rdma.wait_send()

        local_amax = jnp.max(jnp.abs(out_ref[:, :]))
        amax_ref[my, :] = jnp.full((128,), local_amax, jnp.float32)
        amax_sends = []
        for d in range(1, N_DEV):
            dst = lax.rem(my + d, N_DEV)
            rdma = pltpu.make_async_remote_copy(
                src_ref=amax_ref.at[my],
                dst_ref=amax_ref.at[my],
                send_sem=amax_send_sems.at[d],
                recv_sem=amax_recv_sems.at[my],
                device_id=(dst,),
                device_id_type=pl.DeviceIdType.MESH,
            )
            rdma.start()
            amax_sends.append(rdma)
        for d in range(1, N_DEV):
            s = lax.rem(my + d, N_DEV)
            pltpu.make_async_remote_copy(
                src_ref=amax_ref.at[s], dst_ref=amax_ref.at[s],
                send_sem=amax_send_sems.at[d], recv_sem=amax_recv_sems.at[s],
                device_id=(my,), device_id_type=pl.DeviceIdType.MESH,
            ).wait_recv()
        for rdma in amax_sends:
            rdma.wait_send()

        gmax = jnp.max(amax_ref[:, :])
        scale = gmax / 448.0
        q = jnp.clip(out_ref[:, :] / scale, -448.0, 448.0).astype(
            jnp.float8_e4m3fn
        )
        out_ref[:, :] = q.astype(jnp.float32) * scale

    return pl.pallas_call(
        body,
        out_shape=jax.ShapeDtypeStruct((m_per, n), jnp.float32),
        in_specs=[
            pl.BlockSpec(memory_space=pl.ANY),
            pl.BlockSpec(memory_space=pl.ANY),
        ],
        out_specs=pl.BlockSpec(memory_space=pltpu.VMEM),
        scratch_shapes=[
            pltpu.VMEM((4, m_per, k_per), jnp.float32),
            pltpu.VMEM((N_DEV, m_per, k_per), jnp.bfloat16),
            pltpu.VMEM((N_DEV, m_per, k_per), jnp.bfloat16),
            pltpu.VMEM((3, m_per, k_per), jnp.bfloat16),
            pltpu.VMEM((k, n), jnp.float32),
            pltpu.VMEM((N_DEV, 128), jnp.float32),
            pltpu.SemaphoreType.DMA((4,)),
            pltpu.SemaphoreType.DMA((N_DEV,)),
            pltpu.SemaphoreType.DMA((12,)),
            pltpu.SemaphoreType.DMA((N_DEV,)),
            pltpu.SemaphoreType.DMA((3,)),
            pltpu.SemaphoreType.DMA((N_DEV,)),
            pltpu.SemaphoreType.DMA((N_DEV,)),
        ],
        compiler_params=pltpu.CompilerParams(
            collective_id=0, vmem_limit_bytes=100 * 1024 * 1024
        ),
    )(x, w_mat)


# device time: 50582 ns/iter; 1.0794x vs baseline; 1.0794x over previous
import jax
import jax.numpy as jnp
from jax import lax
from jax.experimental import pallas as pl
from jax.experimental.pallas import tpu as pltpu

N_DEV = 8


def kernel(x, w_mat):
    m, k_per = x.shape
    k, n = w_mat.shape
    m_per = m // N_DEV

    def body(x_hbm, w_hbm, out_ref, xland_ref, stage_ref, recv_ref,
             relay_ref, w_vmem, amax_ref, xdma_sems, wdma_sems,
             send_sems, frecv_sems, rrecv_sems,
             amax_send_sems, amax_recv_sems):
        my = lax.axis_index("i")
        p1 = my ^ 1
        p3 = my ^ 3
        p4 = my ^ 4

        barrier_sem = pltpu.get_barrier_semaphore()
        for v in (1, 3, 4):
            pl.semaphore_signal(
                barrier_sem, inc=1,
                device_id=(my ^ v,), device_id_type=pl.DeviceIdType.MESH,
            )

        dests = [my ^ 2, my ^ 7, my ^ 5, my ^ 6, my, p1, p4, p3]

        def xcopy(i):
            return pltpu.make_async_copy(
                x_hbm.at[pl.ds(dests[i] * m_per, m_per), :],
                xland_ref.at[i % 4],
                xdma_sems.at[i % 4],
            )

        def wcopy(s):
            return pltpu.make_async_copy(
                w_hbm.at[pl.ds(s * k_per, k_per), :],
                w_vmem.at[pl.ds(s * k_per, k_per), :],
                wdma_sems.at[s],
            )

        for i in range(4):
            xcopy(i).start()
        for v in (0, 2, 7, 5, 4, 1, 3, 6):
            wcopy(my ^ v).start()

        pl.semaphore_wait(barrier_sem, 3)

        def send(sid, src, dst_dev, dst_ref, sem):
            rdma = pltpu.make_async_remote_copy(
                src_ref=src,
                dst_ref=dst_ref,
                send_sem=send_sems.at[sid],
                recv_sem=sem,
                device_id=(dst_dev,),
                device_id_type=pl.DeviceIdType.MESH,
            )
            rdma.start()
            return rdma

        def wait_relay(j):
            pltpu.make_async_remote_copy(
                src_ref=relay_ref.at[j], dst_ref=relay_ref.at[j],
                send_sem=send_sems.at[0], recv_sem=rrecv_sems.at[j],
                device_id=(my,), device_id_type=pl.DeviceIdType.MESH,
            ).wait_recv()

        def wait_final(v):
            pltpu.make_async_remote_copy(
                src_ref=recv_ref.at[v], dst_ref=recv_ref.at[v],
                send_sem=send_sems.at[0], recv_sem=frecv_sems.at[v],
                device_id=(my,), device_id_type=pl.DeviceIdType.MESH,
            ).wait_recv()

        def gemm(v):
            s = my ^ v
            wcopy(s).wait()
            out_ref[:, :] = out_ref[:, :] + jnp.dot(
                recv_ref[v],
                w_vmem[pl.ds(s * k_per, k_per), :],
                preferred_element_type=jnp.float32,
            )

        def land(i):
            xcopy(i).wait()
            stage_ref[i, :, :] = xland_ref[i % 4].astype(jnp.bfloat16)
            if i + 4 < 8:
                xcopy(i + 4).start()

        sends = []
        land(0)
        sends.append(send(1, stage_ref.at[0], p1,
                          relay_ref.at[0], rrecv_sems.at[0]))
        land(1)
        sends.append(send(2, stage_ref.at[1], p3,
                          relay_ref.at[2], rrecv_sems.at[2]))
        land(2)
        sends.append(send(3, stage_ref.at[2], p4,
                          relay_ref.at[1], rrecv_sems.at[1]))
        land(3)
        sends.append(send(4, stage_ref.at[3], p1,
                          relay_ref.at[3], rrecv_sems.at[3]))
        land(4)

        wcopy(my).wait()
        out_ref[:, :] = jnp.dot(
            stage_ref[4],
            w_vmem[pl.ds(my * k_per, k_per), :],
            preferred_element_type=jnp.float32,
        )

        wait_relay(0)
        sends.append(send(5, relay_ref.at[0], p3,
                          recv_ref.at[2], frecv_sems.at[2]))
        wait_relay(1)
        sends.append(send(6, relay_ref.at[1], p1,
                          recv_ref.at[5], frecv_sems.at[5]))
        wait_relay(2)
        sends.append(send(7, relay_ref.at[2], p4,
                          recv_ref.at[7], frecv_sems.at[7]))

        land(5)
        sends.append(send(8, stage_ref.at[5], p1,
                          recv_ref.at[1], frecv_sems.at[1]))
        land(6)
        sends.append(send(9, stage_ref.at[6], p4,
                          recv_ref.at[4], frecv_sems.at[4]))

        wait_relay(3)
        sends.append(send(10, relay_ref.at[3], p3,
                          relay_ref.at[4], rrecv_sems.at[4]))
        land(7)
        sends.append(send(11, stage_ref.at[7], p3,
                          recv_ref.at[3], frecv_sems.at[3]))

        wait_final(2)
        gemm(2)
        wait_final(7)
        gemm(7)

        wait_relay(4)
        sends.append(send(0, relay_ref.at[4], p4,
                          recv_ref.at[6], frecv_sems.at[6]))

        for v in (5, 4, 1, 3, 6):
            wait_final(v)
            gemm(v)

        for rdma in sends:
            rdma.wait_send()

        local_amax = jnp.max(jnp.abs(out_ref[:, :]))
        amax_ref[my, :] = jnp.full((128,), local_amax, jnp.float32)
        amax_sends = []
        for d in range(1, N_DEV):
            dst = lax.rem(my + d, N_DEV)
            rdma = pltpu.make_async_remote_copy(
                src_ref=amax_ref.at[my],
                dst_ref=amax_ref.at[my],
                send_sem=amax_send_sems.at[d],
                recv_sem=amax_recv_sems.at[my],
                device_id=(dst,),
                device_id_type=pl.DeviceIdType.MESH,
            )
            rdma.start()
            amax_sends.append(rdma)
        for d in range(1, N_DEV):
            s = lax.rem(my + d, N_DEV)
            pltpu.make_async_remote_copy(
                src_ref=amax_ref.at[s], dst_ref=amax_ref.at[s],
                send_sem=amax_send_sems.at[d], recv_sem=amax_recv_sems.at[s],
                device_id=(my,), device_id_type=pl.DeviceIdType.MESH,
            ).wait_recv()
        for rdma in amax_sends:
            rdma.wait_send()

        gmax = jnp.max(amax_ref[:, :])
        scale = gmax / 448.0
        q = (out_ref[:, :] / scale).astype(jnp.float8_e4m3fn)
        out_ref[:, :] = q.astype(jnp.float32) * scale

    return pl.pallas_call(
        body,
        out_shape=jax.ShapeDtypeStruct((m_per, n), jnp.float32),
        in_specs=[
            pl.BlockSpec(memory_space=pl.ANY),
            pl.BlockSpec(memory_space=pl.ANY),
        ],
        out_specs=pl.BlockSpec(memory_space=pltpu.VMEM),
        scratch_shapes=[
            pltpu.VMEM((4, m_per, k_per), jnp.float32),
            pltpu.VMEM((N_DEV, m_per, k_per), jnp.bfloat16),
            pltpu.VMEM((N_DEV, m_per, k_per), jnp.bfloat16),
            pltpu.VMEM((5, m_per, k_per), jnp.bfloat16),
            pltpu.VMEM((k, n), jnp.float32),
            pltpu.VMEM((N_DEV, 128), jnp.float32),
            pltpu.SemaphoreType.DMA((4,)),
            pltpu.SemaphoreType.DMA((N_DEV,)),
            pltpu.SemaphoreType.DMA((12,)),
            pltpu.SemaphoreType.DMA((N_DEV,)),
            pltpu.SemaphoreType.DMA((5,)),
            pltpu.SemaphoreType.DMA((N_DEV,)),
            pltpu.SemaphoreType.DMA((N_DEV,)),
        ],
        compiler_params=pltpu.CompilerParams(
            collective_id=0, vmem_limit_bytes=100 * 1024 * 1024
        ),
    )(x, w_mat)
